# baseline (speedup 1.0000x reference)
import functools

import jax
import jax.numpy as jnp
from jax.experimental import pallas as pl
from jax.experimental.pallas import tpu as pltpu

_INV_TEMP = 10.0
_EPS_SQ = 1e-24
_PACK = 4


def _smog_logits_kernel(xr_ref, gr_ref, out_ref,
                        xb_ref, gs_ref, xsem_ref, *, bm, d):
    i = pl.program_id(0)
    n = pl.num_programs(0)
    rows = bm // _PACK
    k = gs_ref.shape[0]

    @pl.when(i == 0)
    def _():
        pltpu.make_async_copy(
            xr_ref.at[pl.ds(0, rows), :], xb_ref.at[0],
            xsem_ref.at[0]).start()
        gr = gr_ref[...]
        for p in range(_PACK):
            gs_ref[pl.Slice(p, k // _PACK, _PACK), :] = (
                gr[:, p * d:(p + 1) * d])
        g = gs_ref[...]
        gs_ref[...] = g * jax.lax.rsqrt(
            jnp.maximum(jnp.sum(g * g, axis=1, keepdims=True), _EPS_SQ))

    @pl.when(i + 1 < n)
    def _():
        pltpu.make_async_copy(
            xr_ref.at[pl.ds((i + 1) * rows, rows), :],
            xb_ref.at[(i + 1) % 2], xsem_ref.at[(i + 1) % 2]).start()

    pltpu.make_async_copy(
        xr_ref.at[pl.ds(i * rows, rows), :], xb_ref.at[i % 2],
        xsem_ref.at[i % 2]).wait()
    xrb = xb_ref[i % 2]

    r_idx = jax.lax.broadcasted_iota(jnp.int32, (bm, rows), 0)
    j_idx = jax.lax.broadcasted_iota(jnp.int32, (bm, rows), 1)
    p1 = (j_idx == r_idx // _PACK).astype(jnp.float32)
    rl = jax.lax.broadcasted_iota(jnp.int32, (bm, _PACK * d), 0)
    ll = jax.lax.broadcasted_iota(jnp.int32, (bm, _PACK * d), 1)
    msel = (ll // d == rl % _PACK).astype(jnp.float32)
    l2 = jax.lax.broadcasted_iota(jnp.int32, (_PACK * d, d), 0)
    d2 = jax.lax.broadcasted_iota(jnp.int32, (_PACK * d, d), 1)
    sel = (l2 % d == d2).astype(jnp.float32)
    u = jax.lax.dot_general(
        p1, xrb, (((1,), (0,)), ((), ())),
        preferred_element_type=jnp.float32) * msel
    x = jax.lax.dot_general(
        u, sel, (((1,), (0,)), ((), ())),
        preferred_element_type=jnp.float32)

    xs = x * (_INV_TEMP * jax.lax.rsqrt(
        jnp.maximum(jnp.sum(x * x, axis=1, keepdims=True), _EPS_SQ)))
    out_ref[...] = jax.lax.dot_general(
        xs, gs_ref[...], (((1,), (1,)), ((), ())),
        preferred_element_type=jnp.float32)


@functools.partial(jax.jit, static_argnames=("bm",))
def _smog_logits(x, group_features, bm):
    b, d = x.shape
    k, _ = group_features.shape
    bm = min(bm, b)
    xr = x.reshape(b // _PACK, d * _PACK)
    gr = group_features.reshape(k // _PACK, d * _PACK)
    return pl.pallas_call(
        functools.partial(_smog_logits_kernel, bm=bm, d=d),
        grid=(b // bm,),
        in_specs=[
            pl.BlockSpec(memory_space=pl.ANY),
            pl.BlockSpec((k // _PACK, d * _PACK), lambda i: (0, 0)),
        ],
        out_specs=pl.BlockSpec((bm, k), lambda i: (i, 0)),
        out_shape=jax.ShapeDtypeStruct((b, k), jnp.float32),
        scratch_shapes=[
            pltpu.VMEM((2, bm // _PACK, d * _PACK), jnp.float32),
            pltpu.VMEM((k, d), jnp.float32),
            pltpu.SemaphoreType.DMA((2,)),
        ],
        compiler_params=pltpu.CompilerParams(
            dimension_semantics=("arbitrary",)),
    )(xr, gr)


def kernel(x, group_features):
    return _smog_logits(x, group_features, bm=256)


# speedup vs baseline: 1.0422x; 1.0422x over previous
import functools

import jax
import jax.numpy as jnp
from jax.experimental import pallas as pl
from jax.experimental.pallas import tpu as pltpu

_INV_TEMP = 10.0
_EPS_SQ = 1e-24


def _smog_logits_kernel(x_ref, g_ref, out_ref):
    x = x_ref[...]
    g = g_ref[...]
    xs = x * (_INV_TEMP * jax.lax.rsqrt(
        jnp.maximum(jnp.sum(x * x, axis=1, keepdims=True), _EPS_SQ)))
    gs = g * jax.lax.rsqrt(
        jnp.maximum(jnp.sum(g * g, axis=1, keepdims=True), _EPS_SQ))
    out_ref[...] = jax.lax.dot_general(
        xs, gs, (((1,), (1,)), ((), ())),
        preferred_element_type=jnp.float32)


@functools.partial(jax.jit, static_argnames=("bm",))
def _smog_logits(x, group_features, bm):
    b, d = x.shape
    k, _ = group_features.shape
    bm = min(bm, b)
    return pl.pallas_call(
        _smog_logits_kernel,
        grid=(b // bm,),
        in_specs=[
            pl.BlockSpec((bm, d), lambda i: (i, 0)),
            pl.BlockSpec((k, d), lambda i: (0, 0)),
        ],
        out_specs=pl.BlockSpec((bm, k), lambda i: (i, 0)),
        out_shape=jax.ShapeDtypeStruct((b, k), jnp.float32),
        compiler_params=pltpu.CompilerParams(
            dimension_semantics=("arbitrary",)),
    )(x, group_features)


def kernel(x, group_features):
    return _smog_logits(x, group_features, bm=256)
